# baseline (speedup 1.0000x reference)
import functools

import jax
import jax.numpy as jnp
from jax import lax
from jax.experimental import pallas as pl
from jax.experimental.pallas import tpu as pltpu
from jax.experimental.pallas import tpu_sc as plsc

PRIOR_COEF = 0.9
NC, NS = 2, 16
NW = NC * NS
CH = 128

_SC_MESH = plsc.VectorSubcoreMesh(
    core_axis_name="c", subcore_axis_name="s", num_cores=NC, num_subcores=NS)


def _ceil_to(v, m):
    return -(-v // m) * m


def _make_sc_deg(K0, K1, deg_rows, rpt):
    Kmax = max(K0, K1)

    @functools.partial(
        pl.kernel,
        out_type=jax.ShapeDtypeStruct((NC * deg_rows,), jnp.float32),
        mesh=_SC_MESH,
        scratch_types=[
            pltpu.VMEM_SHARED((deg_rows,), jnp.float32),
            pltpu.VMEM((Kmax, CH), jnp.int32),
            pltpu.VMEM((CH,), jnp.float32),
        ],
    )
    def sc_deg(dst_hbm, zeros_hbm, deg_out, deg_sh, idx_v, ones_v):
        c = lax.axis_index("c")
        s = lax.axis_index("s")
        pltpu.sync_copy(zeros_hbm, deg_sh.at[pl.ds(s * rpt, rpt)])
        for i in range(CH // 16):
            ones_v[pl.ds(i * 16, 16)] = jnp.full((16,), 1.0, jnp.float32)
        pltpu.sync_copy(dst_hbm.at[c, s], idx_v)
        plsc.subcore_barrier()

        def chunk(j, carry):
            pltpu.sync_copy(ones_v, deg_sh.at[idx_v.at[j]], add=True)
            return carry

        lax.fori_loop(0, jnp.where(c == 0, K0, K1), chunk, 0)
        plsc.subcore_barrier()
        pltpu.sync_copy(deg_sh.at[pl.ds(s * rpt, rpt)],
                        deg_out.at[pl.ds(c * deg_rows + s * rpt, rpt)])

    return sc_deg


def _make_sc_agg(K0, K1, agg_rows, rpt, dh):
    Kmax = max(K0, K1)

    @functools.partial(
        pl.kernel,
        out_type=jax.ShapeDtypeStruct((NC, agg_rows, dh), jnp.float32),
        mesh=_SC_MESH,
        scratch_types=[
            pltpu.VMEM_SHARED((agg_rows, dh), jnp.float32),
            pltpu.VMEM((Kmax, CH), jnp.int32),
            pltpu.VMEM((Kmax, CH), jnp.int32),
            pltpu.VMEM((CH, dh), jnp.float32),
            pltpu.SemaphoreType.DMA,
        ],
    )
    def sc_agg(h_hbm, src_hbm, dst_hbm, zeros_hbm, agg_out,
               agg_sh, src_v, dst_v, rows_v, sem):
        c = lax.axis_index("c")
        s = lax.axis_index("s")
        pltpu.sync_copy(zeros_hbm, agg_sh.at[pl.ds(s * rpt, rpt)])
        pltpu.sync_copy(src_hbm.at[c, s], src_v)
        pltpu.sync_copy(dst_hbm.at[c, s], dst_v)
        plsc.subcore_barrier()

        def chunk(j, carry):
            pltpu.async_copy(h_hbm.at[src_v.at[j]], rows_v, sem).wait()
            pltpu.sync_copy(rows_v, agg_sh.at[dst_v.at[j]], add=True)
            return carry

        lax.fori_loop(0, jnp.where(c == 0, K0, K1), chunk, 0)
        plsc.subcore_barrier()
        pltpu.sync_copy(agg_sh.at[pl.ds(s * rpt, rpt)],
                        agg_out.at[c, pl.ds(s * rpt, rpt)])

    return sc_agg


def _tc_layer_body(h_ref, aggp_ref, degp_ref, ws_ref, wn_ref, b_ref, wg_ref,
                   pr_ref, w1_ref, b1_ref, w2_ref, b2_ref, wo_ref, bo_ref,
                   out_ref, *, ne, final):
    h = h_ref[...]
    dp = degp_ref[...]
    deg = jnp.maximum(dp[0] + dp[1], 1.0)
    logdeg = jnp.log(1.0 + deg)
    ap = aggp_ref[...]
    agg = (ap[0] + ap[1]) / deg
    z = h @ ws_ref[...] + agg @ wn_ref[...] + b_ref[...]
    z = jnp.maximum(z, 0.0)
    logits = z @ wg_ref[...] + (PRIOR_COEF * logdeg) * pr_ref[...]
    gmax = jnp.max(logits, axis=-1, keepdims=True)
    gate_val = 1.0 / jnp.sum(jnp.exp(logits - gmax), axis=-1, keepdims=True)
    iota = lax.broadcasted_iota(jnp.int32, logits.shape, 1)
    cand = jnp.where(logits == gmax, iota, jnp.int32(127))
    amin = jnp.min(cand, axis=-1, keepdims=True)
    acc = jnp.zeros_like(h)
    for e in range(ne):
        eh = jnp.maximum(z @ w1_ref[e] + b1_ref[e], 0.0)
        eo = eh @ w2_ref[e] + b2_ref[e]
        sel = (amin == jnp.int32(e)).astype(jnp.float32)
        acc = acc + sel * eo
    hn = h + gate_val * acc
    if final:
        out_ref[...] = hn @ wo_ref[...] + bo_ref[...]
    else:
        out_ref[...] = hn


def _make_tc_layer(n, dh, ne, nep, agg_rows, deg_rows, out_dim, final):
    R = 1000
    grid = (n // R,)

    def cmap0(i):
        return (0, 0)

    def cmap3(i):
        return (0, 0, 0)

    in_specs = [
        pl.BlockSpec((R, dh), lambda i: (i, 0)),
        pl.BlockSpec((NC, R, dh), lambda i: (0, i, 0)),
        pl.BlockSpec((NC, R, 1), lambda i: (0, i, 0)),
        pl.BlockSpec((dh, dh), cmap0),
        pl.BlockSpec((dh, dh), cmap0),
        pl.BlockSpec((1, dh), cmap0),
        pl.BlockSpec((dh, nep), cmap0),
        pl.BlockSpec((1, nep), cmap0),
        pl.BlockSpec((ne, dh, dh), cmap3),
        pl.BlockSpec((ne, 1, dh), cmap3),
        pl.BlockSpec((ne, dh, dh), cmap3),
        pl.BlockSpec((ne, 1, dh), cmap3),
        pl.BlockSpec((dh, out_dim), cmap0),
        pl.BlockSpec((1, out_dim), cmap0),
    ]
    return pl.pallas_call(
        functools.partial(_tc_layer_body, ne=ne, final=final),
        grid=grid,
        in_specs=in_specs,
        out_specs=pl.BlockSpec((R, out_dim if final else dh), lambda i: (i, 0)),
        out_shape=jax.ShapeDtypeStruct((n, out_dim if final else dh),
                                       jnp.float32),
    )


def kernel(x, edge_index, W_self, W_nbr, b, Wg, prior, W1, b1, W2, b2,
           W_out, b_out):
    n, dh = x.shape
    e = edge_index.shape[1]
    nl = W_self.shape[0]
    ne = W1.shape[1]
    out_dim = W_out.shape[1]

    rpt_a = _ceil_to(-(-(n + 1) // NS), 8)
    agg_rows = rpt_a * NS
    rpt_d = _ceil_to(-(-(n + 1) // NS), 128)
    deg_rows = rpt_d * NS

    F0 = 0.28
    K0 = max(1, round(F0 * e / (NS * CH)))
    cap0 = NS * K0 * CH
    rem = e - cap0
    K1 = -(-rem // (NS * CH))
    cap1 = NS * K1 * CH
    Kmax = max(K0, K1)

    src = edge_index[0].astype(jnp.int32)
    dst = edge_index[1].astype(jnp.int32)
    pad_src = jnp.zeros((cap1 - rem,), jnp.int32)
    pad_dst = n + (jnp.arange(cap1 - rem, dtype=jnp.int32) % (agg_rows - n))
    src0 = src[:cap0].reshape(NS, K0, CH)
    dst0 = dst[:cap0].reshape(NS, K0, CH)
    src1 = jnp.concatenate([src[cap0:], pad_src]).reshape(NS, K1, CH)
    dst1 = jnp.concatenate([dst[cap0:], pad_dst]).reshape(NS, K1, CH)
    fill_s = jnp.zeros((NS, Kmax - K0, CH), jnp.int32)
    fill_d = jnp.full((NS, Kmax - K0, CH), n, jnp.int32)
    src_p = jnp.stack([jnp.concatenate([src0, fill_s], axis=1), src1])
    dst_p = jnp.stack([jnp.concatenate([dst0, fill_d], axis=1), dst1])
    zeros1d = jnp.zeros((rpt_d,), jnp.float32)
    zeros2d = jnp.zeros((rpt_a, dh), jnp.float32)

    deg_flat = _make_sc_deg(K0, K1, deg_rows, rpt_d)(dst_p, zeros1d)
    degp3 = deg_flat.reshape(NC, deg_rows, 1)

    nep = 8
    b2d = b.reshape(nl, 1, dh)
    b1r = b1.reshape(nl, ne, 1, dh)
    b2r = b2.reshape(nl, ne, 1, dh)
    bo2d = b_out.reshape(1, out_dim)

    sc_agg = _make_sc_agg(K0, K1, agg_rows, rpt_a, dh)
    h = x
    for l in range(nl):
        aggp = sc_agg(h, src_p, dst_p, zeros2d)
        wg_l = jnp.zeros((dh, nep), jnp.float32).at[:, :ne].set(Wg[l])
        pr_l = jnp.full((1, nep), -1e9, jnp.float32).at[0, :ne].set(prior[l])
        final = (l == nl - 1)
        tc = _make_tc_layer(n, dh, ne, nep, agg_rows, deg_rows, out_dim, final)
        h = tc(h, aggp, degp3, W_self[l], W_nbr[l], b2d[l], wg_l, pr_l,
               W1[l], b1r[l], W2[l], b2r[l], W_out, bo2d)
    return h


# speedup vs baseline: 1.1194x; 1.1194x over previous
import functools

import jax
import jax.numpy as jnp
from jax import lax
from jax.experimental import pallas as pl
from jax.experimental.pallas import tpu as pltpu
from jax.experimental.pallas import tpu_sc as plsc

PRIOR_COEF = 0.9
NC, NS = 2, 16
NW = NC * NS
CH = 128

_SC_MESH = plsc.VectorSubcoreMesh(
    core_axis_name="c", subcore_axis_name="s", num_cores=NC, num_subcores=NS)


def _ceil_to(v, m):
    return -(-v // m) * m


def _make_sc_deg(K0, K1, deg_rows, rpt):
    Kmax = max(K0, K1)

    @functools.partial(
        pl.kernel,
        out_type=jax.ShapeDtypeStruct((NC * deg_rows,), jnp.float32),
        mesh=_SC_MESH,
        scratch_types=[
            pltpu.VMEM_SHARED((deg_rows,), jnp.float32),
            pltpu.VMEM((Kmax, CH), jnp.int32),
            pltpu.VMEM((CH,), jnp.float32),
        ],
    )
    def sc_deg(dst_hbm, zeros_hbm, deg_out, deg_sh, idx_v, ones_v):
        c = lax.axis_index("c")
        s = lax.axis_index("s")
        pltpu.sync_copy(zeros_hbm, deg_sh.at[pl.ds(s * rpt, rpt)])
        for i in range(CH // 16):
            ones_v[pl.ds(i * 16, 16)] = jnp.full((16,), 1.0, jnp.float32)
        pltpu.sync_copy(dst_hbm.at[c, s], idx_v)
        plsc.subcore_barrier()

        def chunk(j, carry):
            pltpu.sync_copy(ones_v, deg_sh.at[idx_v.at[j]], add=True)
            return carry

        lax.fori_loop(0, jnp.where(c == 0, K0, K1), chunk, 0)
        plsc.subcore_barrier()
        pltpu.sync_copy(deg_sh.at[pl.ds(s * rpt, rpt)],
                        deg_out.at[pl.ds(c * deg_rows + s * rpt, rpt)])

    return sc_deg


def _make_sc_agg(K0, K1, agg_rows, rpt, dh):
    Kmax = max(K0, K1)

    @functools.partial(
        pl.kernel,
        out_type=jax.ShapeDtypeStruct((NC, agg_rows, dh), jnp.float32),
        mesh=_SC_MESH,
        scratch_types=[
            pltpu.VMEM_SHARED((agg_rows, dh), jnp.float32),
            pltpu.VMEM((Kmax, CH), jnp.int32),
            pltpu.VMEM((Kmax, CH), jnp.int32),
            pltpu.VMEM((CH, dh), jnp.float32),
            pltpu.SemaphoreType.DMA,
        ],
    )
    def sc_agg(h_hbm, src_hbm, dst_hbm, zeros_hbm, agg_out,
               agg_sh, src_v, dst_v, rows_v, sem):
        c = lax.axis_index("c")
        s = lax.axis_index("s")
        pltpu.sync_copy(zeros_hbm, agg_sh.at[pl.ds(s * rpt, rpt)])
        pltpu.sync_copy(src_hbm.at[c, s], src_v)
        pltpu.sync_copy(dst_hbm.at[c, s], dst_v)
        plsc.subcore_barrier()

        def chunk(j, carry):
            pltpu.async_copy(h_hbm.at[src_v.at[j]], rows_v, sem).wait()
            pltpu.sync_copy(rows_v, agg_sh.at[dst_v.at[j]], add=True)
            return carry

        lax.fori_loop(0, jnp.where(c == 0, K0, K1), chunk, 0)
        plsc.subcore_barrier()
        pltpu.sync_copy(agg_sh.at[pl.ds(s * rpt, rpt)],
                        agg_out.at[c, pl.ds(s * rpt, rpt)])

    return sc_agg


def _tc_layer_body(h_ref, aggp_ref, degp_ref, ws_ref, wn_ref, b_ref, wg_ref,
                   pr_ref, w1_ref, b1_ref, w2_ref, b2_ref, wo_ref, bo_ref,
                   out_ref, *, ne, final):
    h = h_ref[...]
    dp = degp_ref[...]
    deg = jnp.maximum(dp[0] + dp[1], 1.0)
    logdeg = jnp.log(1.0 + deg)
    ap = aggp_ref[...]
    agg = (ap[0] + ap[1]) / deg
    z = h @ ws_ref[...] + agg @ wn_ref[...] + b_ref[...]
    z = jnp.maximum(z, 0.0)
    logits = z @ wg_ref[...] + (PRIOR_COEF * logdeg) * pr_ref[...]
    gmax = jnp.max(logits, axis=-1, keepdims=True)
    gate_val = 1.0 / jnp.sum(jnp.exp(logits - gmax), axis=-1, keepdims=True)
    iota = lax.broadcasted_iota(jnp.int32, logits.shape, 1)
    cand = jnp.where(logits == gmax, iota, jnp.int32(127))
    amin = jnp.min(cand, axis=-1, keepdims=True)
    acc = jnp.zeros_like(h)
    for e in range(ne):
        eh = jnp.maximum(z @ w1_ref[e] + b1_ref[e], 0.0)
        eo = eh @ w2_ref[e] + b2_ref[e]
        sel = (amin == jnp.int32(e)).astype(jnp.float32)
        acc = acc + sel * eo
    hn = h + gate_val * acc
    if final:
        out_ref[...] = hn @ wo_ref[...] + bo_ref[...]
    else:
        out_ref[...] = hn


def _make_tc_layer(n, dh, ne, nep, agg_rows, deg_rows, out_dim, final):
    R = 1000
    grid = (n // R,)

    def cmap0(i):
        return (0, 0)

    def cmap3(i):
        return (0, 0, 0)

    in_specs = [
        pl.BlockSpec((R, dh), lambda i: (i, 0)),
        pl.BlockSpec((NC, R, dh), lambda i: (0, i, 0)),
        pl.BlockSpec((NC, R, 1), lambda i: (0, i, 0)),
        pl.BlockSpec((dh, dh), cmap0),
        pl.BlockSpec((dh, dh), cmap0),
        pl.BlockSpec((1, dh), cmap0),
        pl.BlockSpec((dh, nep), cmap0),
        pl.BlockSpec((1, nep), cmap0),
        pl.BlockSpec((ne, dh, dh), cmap3),
        pl.BlockSpec((ne, 1, dh), cmap3),
        pl.BlockSpec((ne, dh, dh), cmap3),
        pl.BlockSpec((ne, 1, dh), cmap3),
        pl.BlockSpec((dh, out_dim), cmap0),
        pl.BlockSpec((1, out_dim), cmap0),
    ]
    return pl.pallas_call(
        functools.partial(_tc_layer_body, ne=ne, final=final),
        grid=grid,
        in_specs=in_specs,
        out_specs=pl.BlockSpec((R, out_dim if final else dh), lambda i: (i, 0)),
        out_shape=jax.ShapeDtypeStruct((n, out_dim if final else dh),
                                       jnp.float32),
    )


def kernel(x, edge_index, W_self, W_nbr, b, Wg, prior, W1, b1, W2, b2,
           W_out, b_out):
    n, dh = x.shape
    e = edge_index.shape[1]
    nl = W_self.shape[0]
    ne = W1.shape[1]
    out_dim = W_out.shape[1]

    rpt_a = _ceil_to(-(-(n + 1) // NS), 8)
    agg_rows = rpt_a * NS
    rpt_d = _ceil_to(-(-(n + 1) // NS), 128)
    deg_rows = rpt_d * NS

    F0 = 0.42
    K0 = max(1, round(F0 * e / (NS * CH)))
    cap0 = NS * K0 * CH
    rem = e - cap0
    K1 = -(-rem // (NS * CH))
    cap1 = NS * K1 * CH
    Kmax = max(K0, K1)

    src = edge_index[0].astype(jnp.int32)
    dst = edge_index[1].astype(jnp.int32)
    pad_src = jnp.zeros((cap1 - rem,), jnp.int32)
    pad_dst = n + (jnp.arange(cap1 - rem, dtype=jnp.int32) % (agg_rows - n))
    src0 = src[:cap0].reshape(NS, K0, CH)
    dst0 = dst[:cap0].reshape(NS, K0, CH)
    src1 = jnp.concatenate([src[cap0:], pad_src]).reshape(NS, K1, CH)
    dst1 = jnp.concatenate([dst[cap0:], pad_dst]).reshape(NS, K1, CH)
    fill_s = jnp.zeros((NS, Kmax - K0, CH), jnp.int32)
    fill_d = jnp.full((NS, Kmax - K0, CH), n, jnp.int32)
    src_p = jnp.stack([jnp.concatenate([src0, fill_s], axis=1), src1])
    dst_p = jnp.stack([jnp.concatenate([dst0, fill_d], axis=1), dst1])
    zeros1d = jnp.zeros((rpt_d,), jnp.float32)
    zeros2d = jnp.zeros((rpt_a, dh), jnp.float32)

    deg_flat = _make_sc_deg(K0, K1, deg_rows, rpt_d)(dst_p, zeros1d)
    degp3 = deg_flat.reshape(NC, deg_rows, 1)

    nep = 8
    b2d = b.reshape(nl, 1, dh)
    b1r = b1.reshape(nl, ne, 1, dh)
    b2r = b2.reshape(nl, ne, 1, dh)
    bo2d = b_out.reshape(1, out_dim)

    sc_agg = _make_sc_agg(K0, K1, agg_rows, rpt_a, dh)
    h = x
    for l in range(nl):
        aggp = sc_agg(h, src_p, dst_p, zeros2d)
        wg_l = jnp.zeros((dh, nep), jnp.float32).at[:, :ne].set(Wg[l])
        pr_l = jnp.full((1, nep), -1e9, jnp.float32).at[0, :ne].set(prior[l])
        final = (l == nl - 1)
        tc = _make_tc_layer(n, dh, ne, nep, agg_rows, deg_rows, out_dim, final)
        h = tc(h, aggp, degp3, W_self[l], W_nbr[l], b2d[l], wg_l, pr_l,
               W1[l], b1r[l], W2[l], b2r[l], W_out, bo2d)
    return h


# speedup vs baseline: 1.1620x; 1.0380x over previous
import functools

import jax
import jax.numpy as jnp
from jax import lax
from jax.experimental import pallas as pl
from jax.experimental.pallas import tpu as pltpu
from jax.experimental.pallas import tpu_sc as plsc

PRIOR_COEF = 0.9
NC, NS = 2, 16
NW = NC * NS
CH = 128

_SC_MESH = plsc.VectorSubcoreMesh(
    core_axis_name="c", subcore_axis_name="s", num_cores=NC, num_subcores=NS)


def _ceil_to(v, m):
    return -(-v // m) * m


def _make_sc_deg(K0, K1, deg_rows, rpt):
    Kmax = max(K0, K1)

    @functools.partial(
        pl.kernel,
        out_type=jax.ShapeDtypeStruct((NC * deg_rows,), jnp.float32),
        mesh=_SC_MESH,
        scratch_types=[
            pltpu.VMEM_SHARED((deg_rows,), jnp.float32),
            pltpu.VMEM((Kmax, CH), jnp.int32),
            pltpu.VMEM((CH,), jnp.float32),
        ],
    )
    def sc_deg(dst_hbm, zeros_hbm, deg_out, deg_sh, idx_v, ones_v):
        c = lax.axis_index("c")
        s = lax.axis_index("s")
        pltpu.sync_copy(zeros_hbm, deg_sh.at[pl.ds(s * rpt, rpt)])
        for i in range(CH // 16):
            ones_v[pl.ds(i * 16, 16)] = jnp.full((16,), 1.0, jnp.float32)
        pltpu.sync_copy(dst_hbm.at[c, s], idx_v)
        plsc.subcore_barrier()

        def chunk(j, carry):
            pltpu.sync_copy(ones_v, deg_sh.at[idx_v.at[j]], add=True)
            return carry

        lax.fori_loop(0, jnp.where(c == 0, K0, K1), chunk, 0)
        plsc.subcore_barrier()
        pltpu.sync_copy(deg_sh.at[pl.ds(s * rpt, rpt)],
                        deg_out.at[pl.ds(c * deg_rows + s * rpt, rpt)])

    return sc_deg


def _make_sc_agg(K0, K1, agg_rows, rpt, dh):
    Kmax = max(K0, K1)

    @functools.partial(
        pl.kernel,
        out_type=jax.ShapeDtypeStruct((NC, agg_rows, dh), jnp.float32),
        mesh=_SC_MESH,
        scratch_types=[
            pltpu.VMEM_SHARED((agg_rows, dh), jnp.float32),
            pltpu.VMEM((Kmax, CH), jnp.int32),
            pltpu.VMEM((Kmax, CH), jnp.int32),
            pltpu.VMEM((CH, dh), jnp.float32),
            pltpu.SemaphoreType.DMA,
        ],
    )
    def sc_agg(h_hbm, src_hbm, dst_hbm, zeros_hbm, agg_out,
               agg_sh, src_v, dst_v, rows_v, sem):
        c = lax.axis_index("c")
        s = lax.axis_index("s")
        pltpu.sync_copy(zeros_hbm, agg_sh.at[pl.ds(s * rpt, rpt)])
        pltpu.sync_copy(src_hbm.at[c, s], src_v)
        pltpu.sync_copy(dst_hbm.at[c, s], dst_v)
        plsc.subcore_barrier()

        def chunk(j, carry):
            pltpu.async_copy(h_hbm.at[src_v.at[j]], rows_v, sem).wait()
            pltpu.sync_copy(rows_v, agg_sh.at[dst_v.at[j]], add=True)
            return carry

        lax.fori_loop(0, jnp.where(c == 0, K0, K1), chunk, 0)
        plsc.subcore_barrier()
        pltpu.sync_copy(agg_sh.at[pl.ds(s * rpt, rpt)],
                        agg_out.at[c, pl.ds(s * rpt, rpt)])

    return sc_agg


def _tc_layer_body(h_ref, aggp_ref, degp_ref, ws_ref, wn_ref, b_ref, wg_ref,
                   pr_ref, w1_ref, b1_ref, w2_ref, b2_ref, wo_ref, bo_ref,
                   out_ref, *, ne, final):
    h = h_ref[...]
    dp = degp_ref[...]
    deg = jnp.maximum(dp[0] + dp[1], 1.0)
    logdeg = jnp.log(1.0 + deg)
    ap = aggp_ref[...]
    agg = (ap[0] + ap[1]) / deg
    z = h @ ws_ref[...] + agg @ wn_ref[...] + b_ref[...]
    z = jnp.maximum(z, 0.0)
    logits = z @ wg_ref[...] + (PRIOR_COEF * logdeg) * pr_ref[...]
    gmax = jnp.max(logits, axis=-1, keepdims=True)
    gate_val = 1.0 / jnp.sum(jnp.exp(logits - gmax), axis=-1, keepdims=True)
    iota = lax.broadcasted_iota(jnp.int32, logits.shape, 1)
    cand = jnp.where(logits == gmax, iota, jnp.int32(127))
    amin = jnp.min(cand, axis=-1, keepdims=True)
    acc = jnp.zeros_like(h)
    for e in range(ne):
        eh = jnp.maximum(z @ w1_ref[e] + b1_ref[e], 0.0)
        eo = eh @ w2_ref[e] + b2_ref[e]
        sel = (amin == jnp.int32(e)).astype(jnp.float32)
        acc = acc + sel * eo
    hn = h + gate_val * acc
    if final:
        out_ref[...] = hn @ wo_ref[...] + bo_ref[...]
    else:
        out_ref[...] = hn


def _make_tc_layer(n, dh, ne, nep, agg_rows, deg_rows, out_dim, final):
    R = 1000
    grid = (n // R,)

    def cmap0(i):
        return (0, 0)

    def cmap3(i):
        return (0, 0, 0)

    in_specs = [
        pl.BlockSpec((R, dh), lambda i: (i, 0)),
        pl.BlockSpec((NC, R, dh), lambda i: (0, i, 0)),
        pl.BlockSpec((NC, R, 1), lambda i: (0, i, 0)),
        pl.BlockSpec((dh, dh), cmap0),
        pl.BlockSpec((dh, dh), cmap0),
        pl.BlockSpec((1, dh), cmap0),
        pl.BlockSpec((dh, nep), cmap0),
        pl.BlockSpec((1, nep), cmap0),
        pl.BlockSpec((ne, dh, dh), cmap3),
        pl.BlockSpec((ne, 1, dh), cmap3),
        pl.BlockSpec((ne, dh, dh), cmap3),
        pl.BlockSpec((ne, 1, dh), cmap3),
        pl.BlockSpec((dh, out_dim), cmap0),
        pl.BlockSpec((1, out_dim), cmap0),
    ]
    return pl.pallas_call(
        functools.partial(_tc_layer_body, ne=ne, final=final),
        grid=grid,
        in_specs=in_specs,
        out_specs=pl.BlockSpec((R, out_dim if final else dh), lambda i: (i, 0)),
        out_shape=jax.ShapeDtypeStruct((n, out_dim if final else dh),
                                       jnp.float32),
    )


def kernel(x, edge_index, W_self, W_nbr, b, Wg, prior, W1, b1, W2, b2,
           W_out, b_out):
    n, dh = x.shape
    e = edge_index.shape[1]
    nl = W_self.shape[0]
    ne = W1.shape[1]
    out_dim = W_out.shape[1]

    rpt_a = _ceil_to(-(-(n + 1) // NS), 8)
    agg_rows = rpt_a * NS
    rpt_d = _ceil_to(-(-(n + 1) // NS), 128)
    deg_rows = rpt_d * NS

    F0 = 0.46
    K0 = max(1, round(F0 * e / (NS * CH)))
    cap0 = NS * K0 * CH
    rem = e - cap0
    K1 = -(-rem // (NS * CH))
    cap1 = NS * K1 * CH
    Kmax = max(K0, K1)

    src = edge_index[0].astype(jnp.int32)
    dst = edge_index[1].astype(jnp.int32)
    pad_src = jnp.zeros((cap1 - rem,), jnp.int32)
    pad_dst = n + (jnp.arange(cap1 - rem, dtype=jnp.int32) % (agg_rows - n))
    src0 = src[:cap0].reshape(NS, K0, CH)
    dst0 = dst[:cap0].reshape(NS, K0, CH)
    src1 = jnp.concatenate([src[cap0:], pad_src]).reshape(NS, K1, CH)
    dst1 = jnp.concatenate([dst[cap0:], pad_dst]).reshape(NS, K1, CH)
    fill_s = jnp.zeros((NS, Kmax - K0, CH), jnp.int32)
    fill_d = jnp.full((NS, Kmax - K0, CH), n, jnp.int32)
    src_p = jnp.stack([jnp.concatenate([src0, fill_s], axis=1), src1])
    dst_p = jnp.stack([jnp.concatenate([dst0, fill_d], axis=1), dst1])
    zeros1d = jnp.zeros((rpt_d,), jnp.float32)
    zeros2d = jnp.zeros((rpt_a, dh), jnp.float32)

    deg_flat = _make_sc_deg(K0, K1, deg_rows, rpt_d)(dst_p, zeros1d)
    degp3 = deg_flat.reshape(NC, deg_rows, 1)

    nep = 8
    b2d = b.reshape(nl, 1, dh)
    b1r = b1.reshape(nl, ne, 1, dh)
    b2r = b2.reshape(nl, ne, 1, dh)
    bo2d = b_out.reshape(1, out_dim)

    sc_agg = _make_sc_agg(K0, K1, agg_rows, rpt_a, dh)
    h = x
    for l in range(nl):
        aggp = sc_agg(h, src_p, dst_p, zeros2d)
        wg_l = jnp.zeros((dh, nep), jnp.float32).at[:, :ne].set(Wg[l])
        pr_l = jnp.full((1, nep), -1e9, jnp.float32).at[0, :ne].set(prior[l])
        final = (l == nl - 1)
        tc = _make_tc_layer(n, dh, ne, nep, agg_rows, deg_rows, out_dim, final)
        h = tc(h, aggp, degp3, W_self[l], W_nbr[l], b2d[l], wg_l, pr_l,
               W1[l], b1r[l], W2[l], b2r[l], W_out, bo2d)
    return h


# speedup vs baseline: 1.2059x; 1.0378x over previous
import functools

import jax
import jax.numpy as jnp
from jax import lax
from jax.experimental import pallas as pl
from jax.experimental.pallas import tpu as pltpu
from jax.experimental.pallas import tpu_sc as plsc

PRIOR_COEF = 0.9
NC, NS = 2, 16
NW = NC * NS
CH = 128

_SC_MESH = plsc.VectorSubcoreMesh(
    core_axis_name="c", subcore_axis_name="s", num_cores=NC, num_subcores=NS)


def _ceil_to(v, m):
    return -(-v // m) * m


def _make_sc_deg(K0, K1, deg_rows, rpt):
    Kmax = max(K0, K1)

    @functools.partial(
        pl.kernel,
        out_type=jax.ShapeDtypeStruct((NC * deg_rows,), jnp.float32),
        mesh=_SC_MESH,
        scratch_types=[
            pltpu.VMEM_SHARED((deg_rows,), jnp.float32),
            pltpu.VMEM((Kmax, CH), jnp.int32),
            pltpu.VMEM((CH,), jnp.float32),
        ],
    )
    def sc_deg(dst_hbm, zeros_hbm, deg_out, deg_sh, idx_v, ones_v):
        c = lax.axis_index("c")
        s = lax.axis_index("s")
        pltpu.sync_copy(zeros_hbm, deg_sh.at[pl.ds(s * rpt, rpt)])
        for i in range(CH // 16):
            ones_v[pl.ds(i * 16, 16)] = jnp.full((16,), 1.0, jnp.float32)
        pltpu.sync_copy(dst_hbm.at[c, s], idx_v)
        plsc.subcore_barrier()

        def chunk(j, carry):
            pltpu.sync_copy(ones_v, deg_sh.at[idx_v.at[j]], add=True)
            return carry

        lax.fori_loop(0, jnp.where(c == 0, K0, K1), chunk, 0)
        plsc.subcore_barrier()
        pltpu.sync_copy(deg_sh.at[pl.ds(s * rpt, rpt)],
                        deg_out.at[pl.ds(c * deg_rows + s * rpt, rpt)])

    return sc_deg


def _make_sc_agg(K0, K1, agg_rows, rpt, dh):
    Kmax = max(K0, K1)

    @functools.partial(
        pl.kernel,
        out_type=jax.ShapeDtypeStruct((NC, agg_rows, dh), jnp.float32),
        mesh=_SC_MESH,
        scratch_types=[
            pltpu.VMEM_SHARED((agg_rows, dh), jnp.float32),
            pltpu.VMEM((Kmax, CH), jnp.int32),
            pltpu.VMEM((Kmax, CH), jnp.int32),
            pltpu.VMEM((CH, dh), jnp.float32),
            pltpu.SemaphoreType.DMA,
        ],
    )
    def sc_agg(h_hbm, src_hbm, dst_hbm, zeros_hbm, agg_out,
               agg_sh, src_v, dst_v, rows_v, sem):
        c = lax.axis_index("c")
        s = lax.axis_index("s")
        pltpu.sync_copy(zeros_hbm, agg_sh.at[pl.ds(s * rpt, rpt)])
        pltpu.sync_copy(src_hbm.at[c, s], src_v)
        pltpu.sync_copy(dst_hbm.at[c, s], dst_v)
        plsc.subcore_barrier()

        def chunk(j, carry):
            pltpu.async_copy(h_hbm.at[src_v.at[j]], rows_v, sem).wait()
            pltpu.sync_copy(rows_v, agg_sh.at[dst_v.at[j]], add=True)
            return carry

        lax.fori_loop(0, jnp.where(c == 0, K0, K1), chunk, 0)
        plsc.subcore_barrier()
        pltpu.sync_copy(agg_sh.at[pl.ds(s * rpt, rpt)],
                        agg_out.at[c, pl.ds(s * rpt, rpt)])

    return sc_agg


def _tc_layer_body(h_ref, aggp_ref, degp_ref, ws_ref, wn_ref, b_ref, wg_ref,
                   pr_ref, w1_ref, b1_ref, w2_ref, b2_ref, wo_ref, bo_ref,
                   out_ref, *, ne, final):
    h = h_ref[...]
    dp = degp_ref[...]
    deg = jnp.maximum(dp[0] + dp[1], 1.0)
    logdeg = jnp.log(1.0 + deg)
    ap = aggp_ref[...]
    agg = (ap[0] + ap[1]) / deg
    z = h @ ws_ref[...] + agg @ wn_ref[...] + b_ref[...]
    z = jnp.maximum(z, 0.0)
    logits = z @ wg_ref[...] + (PRIOR_COEF * logdeg) * pr_ref[...]
    gmax = jnp.max(logits, axis=-1, keepdims=True)
    gate_val = 1.0 / jnp.sum(jnp.exp(logits - gmax), axis=-1, keepdims=True)
    iota = lax.broadcasted_iota(jnp.int32, logits.shape, 1)
    cand = jnp.where(logits == gmax, iota, jnp.int32(127))
    amin = jnp.min(cand, axis=-1, keepdims=True)
    acc = jnp.zeros_like(h)
    for e in range(ne):
        eh = jnp.maximum(z @ w1_ref[e] + b1_ref[e], 0.0)
        eo = eh @ w2_ref[e] + b2_ref[e]
        sel = (amin == jnp.int32(e)).astype(jnp.float32)
        acc = acc + sel * eo
    hn = h + gate_val * acc
    if final:
        out_ref[...] = hn @ wo_ref[...] + bo_ref[...]
    else:
        out_ref[...] = hn


def _make_tc_layer(n, dh, ne, nep, agg_rows, deg_rows, out_dim, final):
    R = 1000
    grid = (n // R,)

    def cmap0(i):
        return (0, 0)

    def cmap3(i):
        return (0, 0, 0)

    in_specs = [
        pl.BlockSpec((R, dh), lambda i: (i, 0)),
        pl.BlockSpec((NC, R, dh), lambda i: (0, i, 0)),
        pl.BlockSpec((NC, R, 1), lambda i: (0, i, 0)),
        pl.BlockSpec((dh, dh), cmap0),
        pl.BlockSpec((dh, dh), cmap0),
        pl.BlockSpec((1, dh), cmap0),
        pl.BlockSpec((dh, nep), cmap0),
        pl.BlockSpec((1, nep), cmap0),
        pl.BlockSpec((ne, dh, dh), cmap3),
        pl.BlockSpec((ne, 1, dh), cmap3),
        pl.BlockSpec((ne, dh, dh), cmap3),
        pl.BlockSpec((ne, 1, dh), cmap3),
        pl.BlockSpec((dh, out_dim), cmap0),
        pl.BlockSpec((1, out_dim), cmap0),
    ]
    return pl.pallas_call(
        functools.partial(_tc_layer_body, ne=ne, final=final),
        grid=grid,
        in_specs=in_specs,
        out_specs=pl.BlockSpec((R, out_dim if final else dh), lambda i: (i, 0)),
        out_shape=jax.ShapeDtypeStruct((n, out_dim if final else dh),
                                       jnp.float32),
    )


def kernel(x, edge_index, W_self, W_nbr, b, Wg, prior, W1, b1, W2, b2,
           W_out, b_out):
    n, dh = x.shape
    e = edge_index.shape[1]
    nl = W_self.shape[0]
    ne = W1.shape[1]
    out_dim = W_out.shape[1]

    rpt_a = _ceil_to(-(-(n + 1) // NS), 8)
    agg_rows = rpt_a * NS
    rpt_d = _ceil_to(-(-(n + 1) // NS), 128)
    deg_rows = rpt_d * NS

    F0 = 0.50
    K0 = max(1, round(F0 * e / (NS * CH)))
    cap0 = NS * K0 * CH
    rem = e - cap0
    K1 = -(-rem // (NS * CH))
    cap1 = NS * K1 * CH
    Kmax = max(K0, K1)

    src = edge_index[0].astype(jnp.int32)
    dst = edge_index[1].astype(jnp.int32)
    pad_src = jnp.zeros((cap1 - rem,), jnp.int32)
    pad_dst = n + (jnp.arange(cap1 - rem, dtype=jnp.int32) % (agg_rows - n))
    src0 = src[:cap0].reshape(NS, K0, CH)
    dst0 = dst[:cap0].reshape(NS, K0, CH)
    src1 = jnp.concatenate([src[cap0:], pad_src]).reshape(NS, K1, CH)
    dst1 = jnp.concatenate([dst[cap0:], pad_dst]).reshape(NS, K1, CH)
    fill_s = jnp.zeros((NS, Kmax - K0, CH), jnp.int32)
    fill_d = jnp.full((NS, Kmax - K0, CH), n, jnp.int32)
    src_p = jnp.stack([jnp.concatenate([src0, fill_s], axis=1), src1])
    dst_p = jnp.stack([jnp.concatenate([dst0, fill_d], axis=1), dst1])
    zeros1d = jnp.zeros((rpt_d,), jnp.float32)
    zeros2d = jnp.zeros((rpt_a, dh), jnp.float32)

    deg_flat = _make_sc_deg(K0, K1, deg_rows, rpt_d)(dst_p, zeros1d)
    degp3 = deg_flat.reshape(NC, deg_rows, 1)

    nep = 8
    b2d = b.reshape(nl, 1, dh)
    b1r = b1.reshape(nl, ne, 1, dh)
    b2r = b2.reshape(nl, ne, 1, dh)
    bo2d = b_out.reshape(1, out_dim)

    sc_agg = _make_sc_agg(K0, K1, agg_rows, rpt_a, dh)
    h = x
    for l in range(nl):
        aggp = sc_agg(h, src_p, dst_p, zeros2d)
        wg_l = jnp.zeros((dh, nep), jnp.float32).at[:, :ne].set(Wg[l])
        pr_l = jnp.full((1, nep), -1e9, jnp.float32).at[0, :ne].set(prior[l])
        final = (l == nl - 1)
        tc = _make_tc_layer(n, dh, ne, nep, agg_rows, deg_rows, out_dim, final)
        h = tc(h, aggp, degp3, W_self[l], W_nbr[l], b2d[l], wg_l, pr_l,
               W1[l], b1r[l], W2[l], b2r[l], W_out, bo2d)
    return h


# speedup vs baseline: 1.2516x; 1.0379x over previous
import functools

import jax
import jax.numpy as jnp
from jax import lax
from jax.experimental import pallas as pl
from jax.experimental.pallas import tpu as pltpu
from jax.experimental.pallas import tpu_sc as plsc

PRIOR_COEF = 0.9
NC, NS = 2, 16
NW = NC * NS
CH = 128

_SC_MESH = plsc.VectorSubcoreMesh(
    core_axis_name="c", subcore_axis_name="s", num_cores=NC, num_subcores=NS)


def _ceil_to(v, m):
    return -(-v // m) * m


def _make_sc_deg(K0, K1, deg_rows, rpt):
    Kmax = max(K0, K1)

    @functools.partial(
        pl.kernel,
        out_type=jax.ShapeDtypeStruct((NC * deg_rows,), jnp.float32),
        mesh=_SC_MESH,
        scratch_types=[
            pltpu.VMEM_SHARED((deg_rows,), jnp.float32),
            pltpu.VMEM((Kmax, CH), jnp.int32),
            pltpu.VMEM((CH,), jnp.float32),
        ],
    )
    def sc_deg(dst_hbm, zeros_hbm, deg_out, deg_sh, idx_v, ones_v):
        c = lax.axis_index("c")
        s = lax.axis_index("s")
        pltpu.sync_copy(zeros_hbm, deg_sh.at[pl.ds(s * rpt, rpt)])
        for i in range(CH // 16):
            ones_v[pl.ds(i * 16, 16)] = jnp.full((16,), 1.0, jnp.float32)
        pltpu.sync_copy(dst_hbm.at[c, s], idx_v)
        plsc.subcore_barrier()

        def chunk(j, carry):
            pltpu.sync_copy(ones_v, deg_sh.at[idx_v.at[j]], add=True)
            return carry

        lax.fori_loop(0, jnp.where(c == 0, K0, K1), chunk, 0)
        plsc.subcore_barrier()
        pltpu.sync_copy(deg_sh.at[pl.ds(s * rpt, rpt)],
                        deg_out.at[pl.ds(c * deg_rows + s * rpt, rpt)])

    return sc_deg


def _make_sc_agg(K0, K1, agg_rows, rpt, dh):
    Kmax = max(K0, K1)

    @functools.partial(
        pl.kernel,
        out_type=jax.ShapeDtypeStruct((NC, agg_rows, dh), jnp.float32),
        mesh=_SC_MESH,
        scratch_types=[
            pltpu.VMEM_SHARED((agg_rows, dh), jnp.float32),
            pltpu.VMEM((Kmax, CH), jnp.int32),
            pltpu.VMEM((Kmax, CH), jnp.int32),
            pltpu.VMEM((CH, dh), jnp.float32),
            pltpu.SemaphoreType.DMA,
        ],
    )
    def sc_agg(h_hbm, src_hbm, dst_hbm, zeros_hbm, agg_out,
               agg_sh, src_v, dst_v, rows_v, sem):
        c = lax.axis_index("c")
        s = lax.axis_index("s")
        pltpu.sync_copy(zeros_hbm, agg_sh.at[pl.ds(s * rpt, rpt)])
        pltpu.sync_copy(src_hbm.at[c, s], src_v)
        pltpu.sync_copy(dst_hbm.at[c, s], dst_v)
        plsc.subcore_barrier()

        def chunk(j, carry):
            pltpu.async_copy(h_hbm.at[src_v.at[j]], rows_v, sem).wait()
            pltpu.sync_copy(rows_v, agg_sh.at[dst_v.at[j]], add=True)
            return carry

        lax.fori_loop(0, jnp.where(c == 0, K0, K1), chunk, 0)
        plsc.subcore_barrier()
        pltpu.sync_copy(agg_sh.at[pl.ds(s * rpt, rpt)],
                        agg_out.at[c, pl.ds(s * rpt, rpt)])

    return sc_agg


def _tc_layer_body(h_ref, aggp_ref, degp_ref, ws_ref, wn_ref, b_ref, wg_ref,
                   pr_ref, w1_ref, b1_ref, w2_ref, b2_ref, wo_ref, bo_ref,
                   out_ref, *, ne, final):
    h = h_ref[...]
    dp = degp_ref[...]
    deg = jnp.maximum(dp[0] + dp[1], 1.0)
    logdeg = jnp.log(1.0 + deg)
    ap = aggp_ref[...]
    agg = (ap[0] + ap[1]) / deg
    z = h @ ws_ref[...] + agg @ wn_ref[...] + b_ref[...]
    z = jnp.maximum(z, 0.0)
    logits = z @ wg_ref[...] + (PRIOR_COEF * logdeg) * pr_ref[...]
    gmax = jnp.max(logits, axis=-1, keepdims=True)
    gate_val = 1.0 / jnp.sum(jnp.exp(logits - gmax), axis=-1, keepdims=True)
    iota = lax.broadcasted_iota(jnp.int32, logits.shape, 1)
    cand = jnp.where(logits == gmax, iota, jnp.int32(127))
    amin = jnp.min(cand, axis=-1, keepdims=True)
    acc = jnp.zeros_like(h)
    for e in range(ne):
        eh = jnp.maximum(z @ w1_ref[e] + b1_ref[e], 0.0)
        eo = eh @ w2_ref[e] + b2_ref[e]
        sel = (amin == jnp.int32(e)).astype(jnp.float32)
        acc = acc + sel * eo
    hn = h + gate_val * acc
    if final:
        out_ref[...] = hn @ wo_ref[...] + bo_ref[...]
    else:
        out_ref[...] = hn


def _make_tc_layer(n, dh, ne, nep, agg_rows, deg_rows, out_dim, final):
    R = 1000
    grid = (n // R,)

    def cmap0(i):
        return (0, 0)

    def cmap3(i):
        return (0, 0, 0)

    in_specs = [
        pl.BlockSpec((R, dh), lambda i: (i, 0)),
        pl.BlockSpec((NC, R, dh), lambda i: (0, i, 0)),
        pl.BlockSpec((NC, R, 1), lambda i: (0, i, 0)),
        pl.BlockSpec((dh, dh), cmap0),
        pl.BlockSpec((dh, dh), cmap0),
        pl.BlockSpec((1, dh), cmap0),
        pl.BlockSpec((dh, nep), cmap0),
        pl.BlockSpec((1, nep), cmap0),
        pl.BlockSpec((ne, dh, dh), cmap3),
        pl.BlockSpec((ne, 1, dh), cmap3),
        pl.BlockSpec((ne, dh, dh), cmap3),
        pl.BlockSpec((ne, 1, dh), cmap3),
        pl.BlockSpec((dh, out_dim), cmap0),
        pl.BlockSpec((1, out_dim), cmap0),
    ]
    return pl.pallas_call(
        functools.partial(_tc_layer_body, ne=ne, final=final),
        grid=grid,
        in_specs=in_specs,
        out_specs=pl.BlockSpec((R, out_dim if final else dh), lambda i: (i, 0)),
        out_shape=jax.ShapeDtypeStruct((n, out_dim if final else dh),
                                       jnp.float32),
    )


def kernel(x, edge_index, W_self, W_nbr, b, Wg, prior, W1, b1, W2, b2,
           W_out, b_out):
    n, dh = x.shape
    e = edge_index.shape[1]
    nl = W_self.shape[0]
    ne = W1.shape[1]
    out_dim = W_out.shape[1]

    rpt_a = _ceil_to(-(-(n + 1) // NS), 8)
    agg_rows = rpt_a * NS
    rpt_d = _ceil_to(-(-(n + 1) // NS), 128)
    deg_rows = rpt_d * NS

    F0 = 0.54
    K0 = max(1, round(F0 * e / (NS * CH)))
    cap0 = NS * K0 * CH
    rem = e - cap0
    K1 = -(-rem // (NS * CH))
    cap1 = NS * K1 * CH
    Kmax = max(K0, K1)

    src = edge_index[0].astype(jnp.int32)
    dst = edge_index[1].astype(jnp.int32)
    pad_src = jnp.zeros((cap1 - rem,), jnp.int32)
    pad_dst = n + (jnp.arange(cap1 - rem, dtype=jnp.int32) % (agg_rows - n))
    src0 = src[:cap0].reshape(NS, K0, CH)
    dst0 = dst[:cap0].reshape(NS, K0, CH)
    src1 = jnp.concatenate([src[cap0:], pad_src]).reshape(NS, K1, CH)
    dst1 = jnp.concatenate([dst[cap0:], pad_dst]).reshape(NS, K1, CH)
    def _pad_chunks(a, k, fillval):
        if k == Kmax:
            return a
        return jnp.concatenate(
            [a, jnp.full((NS, Kmax - k, CH), fillval, jnp.int32)], axis=1)

    src_p = jnp.stack([_pad_chunks(src0, K0, 0), _pad_chunks(src1, K1, 0)])
    dst_p = jnp.stack([_pad_chunks(dst0, K0, n), _pad_chunks(dst1, K1, n)])
    zeros1d = jnp.zeros((rpt_d,), jnp.float32)
    zeros2d = jnp.zeros((rpt_a, dh), jnp.float32)

    deg_flat = _make_sc_deg(K0, K1, deg_rows, rpt_d)(dst_p, zeros1d)
    degp3 = deg_flat.reshape(NC, deg_rows, 1)

    nep = 8
    b2d = b.reshape(nl, 1, dh)
    b1r = b1.reshape(nl, ne, 1, dh)
    b2r = b2.reshape(nl, ne, 1, dh)
    bo2d = b_out.reshape(1, out_dim)

    sc_agg = _make_sc_agg(K0, K1, agg_rows, rpt_a, dh)
    h = x
    for l in range(nl):
        aggp = sc_agg(h, src_p, dst_p, zeros2d)
        wg_l = jnp.zeros((dh, nep), jnp.float32).at[:, :ne].set(Wg[l])
        pr_l = jnp.full((1, nep), -1e9, jnp.float32).at[0, :ne].set(prior[l])
        final = (l == nl - 1)
        tc = _make_tc_layer(n, dh, ne, nep, agg_rows, deg_rows, out_dim, final)
        h = tc(h, aggp, degp3, W_self[l], W_nbr[l], b2d[l], wg_l, pr_l,
               W1[l], b1r[l], W2[l], b2r[l], W_out, bo2d)
    return h


# speedup vs baseline: 1.2999x; 1.0386x over previous
import functools

import jax
import jax.numpy as jnp
from jax import lax
from jax.experimental import pallas as pl
from jax.experimental.pallas import tpu as pltpu
from jax.experimental.pallas import tpu_sc as plsc

PRIOR_COEF = 0.9
NC, NS = 2, 16
NW = NC * NS
CH = 128

_SC_MESH = plsc.VectorSubcoreMesh(
    core_axis_name="c", subcore_axis_name="s", num_cores=NC, num_subcores=NS)


def _ceil_to(v, m):
    return -(-v // m) * m


def _make_sc_deg(K0, K1, deg_rows, rpt):
    Kmax = max(K0, K1)

    @functools.partial(
        pl.kernel,
        out_type=jax.ShapeDtypeStruct((NC * deg_rows,), jnp.float32),
        mesh=_SC_MESH,
        scratch_types=[
            pltpu.VMEM_SHARED((deg_rows,), jnp.float32),
            pltpu.VMEM((Kmax, CH), jnp.int32),
            pltpu.VMEM((CH,), jnp.float32),
        ],
    )
    def sc_deg(dst_hbm, zeros_hbm, deg_out, deg_sh, idx_v, ones_v):
        c = lax.axis_index("c")
        s = lax.axis_index("s")
        pltpu.sync_copy(zeros_hbm, deg_sh.at[pl.ds(s * rpt, rpt)])
        for i in range(CH // 16):
            ones_v[pl.ds(i * 16, 16)] = jnp.full((16,), 1.0, jnp.float32)
        pltpu.sync_copy(dst_hbm.at[c, s], idx_v)
        plsc.subcore_barrier()

        def chunk(j, carry):
            pltpu.sync_copy(ones_v, deg_sh.at[idx_v.at[j]], add=True)
            return carry

        lax.fori_loop(0, jnp.where(c == 0, K0, K1), chunk, 0)
        plsc.subcore_barrier()
        pltpu.sync_copy(deg_sh.at[pl.ds(s * rpt, rpt)],
                        deg_out.at[pl.ds(c * deg_rows + s * rpt, rpt)])

    return sc_deg


def _make_sc_agg(K0, K1, agg_rows, rpt, dh):
    Kmax = max(K0, K1)

    @functools.partial(
        pl.kernel,
        out_type=jax.ShapeDtypeStruct((NC, agg_rows, dh), jnp.float32),
        mesh=_SC_MESH,
        scratch_types=[
            pltpu.VMEM_SHARED((agg_rows, dh), jnp.float32),
            pltpu.VMEM((Kmax, CH), jnp.int32),
            pltpu.VMEM((Kmax, CH), jnp.int32),
            pltpu.VMEM((CH, dh), jnp.float32),
            pltpu.SemaphoreType.DMA,
        ],
    )
    def sc_agg(h_hbm, src_hbm, dst_hbm, zeros_hbm, agg_out,
               agg_sh, src_v, dst_v, rows_v, sem):
        c = lax.axis_index("c")
        s = lax.axis_index("s")
        pltpu.sync_copy(zeros_hbm, agg_sh.at[pl.ds(s * rpt, rpt)])
        pltpu.sync_copy(src_hbm.at[c, s], src_v)
        pltpu.sync_copy(dst_hbm.at[c, s], dst_v)
        plsc.subcore_barrier()

        def chunk(j, carry):
            pltpu.async_copy(h_hbm.at[src_v.at[j]], rows_v, sem).wait()
            pltpu.sync_copy(rows_v, agg_sh.at[dst_v.at[j]], add=True)
            return carry

        lax.fori_loop(0, jnp.where(c == 0, K0, K1), chunk, 0)
        plsc.subcore_barrier()
        pltpu.sync_copy(agg_sh.at[pl.ds(s * rpt, rpt)],
                        agg_out.at[c, pl.ds(s * rpt, rpt)])

    return sc_agg


def _tc_layer_body(h_ref, aggp_ref, degp_ref, ws_ref, wn_ref, b_ref, wg_ref,
                   pr_ref, w1_ref, b1_ref, w2_ref, b2_ref, wo_ref, bo_ref,
                   out_ref, *, ne, final):
    h = h_ref[...]
    dp = degp_ref[...]
    deg = jnp.maximum(dp[0] + dp[1], 1.0)
    logdeg = jnp.log(1.0 + deg)
    ap = aggp_ref[...]
    agg = (ap[0] + ap[1]) / deg
    z = h @ ws_ref[...] + agg @ wn_ref[...] + b_ref[...]
    z = jnp.maximum(z, 0.0)
    logits = z @ wg_ref[...] + (PRIOR_COEF * logdeg) * pr_ref[...]
    gmax = jnp.max(logits, axis=-1, keepdims=True)
    gate_val = 1.0 / jnp.sum(jnp.exp(logits - gmax), axis=-1, keepdims=True)
    iota = lax.broadcasted_iota(jnp.int32, logits.shape, 1)
    cand = jnp.where(logits == gmax, iota, jnp.int32(127))
    amin = jnp.min(cand, axis=-1, keepdims=True)
    acc = jnp.zeros_like(h)
    for e in range(ne):
        eh = jnp.maximum(z @ w1_ref[e] + b1_ref[e], 0.0)
        eo = eh @ w2_ref[e] + b2_ref[e]
        sel = (amin == jnp.int32(e)).astype(jnp.float32)
        acc = acc + sel * eo
    hn = h + gate_val * acc
    if final:
        out_ref[...] = hn @ wo_ref[...] + bo_ref[...]
    else:
        out_ref[...] = hn


def _make_tc_layer(n, dh, ne, nep, agg_rows, deg_rows, out_dim, final):
    R = 1000
    grid = (n // R,)

    def cmap0(i):
        return (0, 0)

    def cmap3(i):
        return (0, 0, 0)

    in_specs = [
        pl.BlockSpec((R, dh), lambda i: (i, 0)),
        pl.BlockSpec((NC, R, dh), lambda i: (0, i, 0)),
        pl.BlockSpec((NC, R, 1), lambda i: (0, i, 0)),
        pl.BlockSpec((dh, dh), cmap0),
        pl.BlockSpec((dh, dh), cmap0),
        pl.BlockSpec((1, dh), cmap0),
        pl.BlockSpec((dh, nep), cmap0),
        pl.BlockSpec((1, nep), cmap0),
        pl.BlockSpec((ne, dh, dh), cmap3),
        pl.BlockSpec((ne, 1, dh), cmap3),
        pl.BlockSpec((ne, dh, dh), cmap3),
        pl.BlockSpec((ne, 1, dh), cmap3),
        pl.BlockSpec((dh, out_dim), cmap0),
        pl.BlockSpec((1, out_dim), cmap0),
    ]
    return pl.pallas_call(
        functools.partial(_tc_layer_body, ne=ne, final=final),
        grid=grid,
        in_specs=in_specs,
        out_specs=pl.BlockSpec((R, out_dim if final else dh), lambda i: (i, 0)),
        out_shape=jax.ShapeDtypeStruct((n, out_dim if final else dh),
                                       jnp.float32),
    )


def kernel(x, edge_index, W_self, W_nbr, b, Wg, prior, W1, b1, W2, b2,
           W_out, b_out):
    n, dh = x.shape
    e = edge_index.shape[1]
    nl = W_self.shape[0]
    ne = W1.shape[1]
    out_dim = W_out.shape[1]

    rpt_a = _ceil_to(-(-(n + 1) // NS), 8)
    agg_rows = rpt_a * NS
    rpt_d = _ceil_to(-(-(n + 1) // NS), 128)
    deg_rows = rpt_d * NS

    F0 = 0.60
    K0 = max(1, round(F0 * e / (NS * CH)))
    cap0 = NS * K0 * CH
    rem = e - cap0
    K1 = -(-rem // (NS * CH))
    cap1 = NS * K1 * CH
    Kmax = max(K0, K1)

    src = edge_index[0].astype(jnp.int32)
    dst = edge_index[1].astype(jnp.int32)
    pad_src = jnp.zeros((cap1 - rem,), jnp.int32)
    pad_dst = n + (jnp.arange(cap1 - rem, dtype=jnp.int32) % (agg_rows - n))
    src0 = src[:cap0].reshape(NS, K0, CH)
    dst0 = dst[:cap0].reshape(NS, K0, CH)
    src1 = jnp.concatenate([src[cap0:], pad_src]).reshape(NS, K1, CH)
    dst1 = jnp.concatenate([dst[cap0:], pad_dst]).reshape(NS, K1, CH)
    def _pad_chunks(a, k, fillval):
        if k == Kmax:
            return a
        return jnp.concatenate(
            [a, jnp.full((NS, Kmax - k, CH), fillval, jnp.int32)], axis=1)

    src_p = jnp.stack([_pad_chunks(src0, K0, 0), _pad_chunks(src1, K1, 0)])
    dst_p = jnp.stack([_pad_chunks(dst0, K0, n), _pad_chunks(dst1, K1, n)])
    zeros1d = jnp.zeros((rpt_d,), jnp.float32)
    zeros2d = jnp.zeros((rpt_a, dh), jnp.float32)

    deg_flat = _make_sc_deg(K0, K1, deg_rows, rpt_d)(dst_p, zeros1d)
    degp3 = deg_flat.reshape(NC, deg_rows, 1)

    nep = 8
    b2d = b.reshape(nl, 1, dh)
    b1r = b1.reshape(nl, ne, 1, dh)
    b2r = b2.reshape(nl, ne, 1, dh)
    bo2d = b_out.reshape(1, out_dim)

    sc_agg = _make_sc_agg(K0, K1, agg_rows, rpt_a, dh)
    h = x
    for l in range(nl):
        aggp = sc_agg(h, src_p, dst_p, zeros2d)
        wg_l = jnp.zeros((dh, nep), jnp.float32).at[:, :ne].set(Wg[l])
        pr_l = jnp.full((1, nep), -1e9, jnp.float32).at[0, :ne].set(prior[l])
        final = (l == nl - 1)
        tc = _make_tc_layer(n, dh, ne, nep, agg_rows, deg_rows, out_dim, final)
        h = tc(h, aggp, degp3, W_self[l], W_nbr[l], b2d[l], wg_l, pr_l,
               W1[l], b1r[l], W2[l], b2r[l], W_out, bo2d)
    return h
